# baseline (speedup 1.0000x reference)
import functools

import jax
import jax.numpy as jnp
import numpy as np
from jax import lax
from jax.experimental import pallas as pl
from jax.experimental.pallas import tpu as pltpu
from jax.experimental.pallas import tpu_sc as plsc

_SLOW_FRAMES = 8
_NUM_CORES = 2
_NUM_SUBCORES = 16


def _select_const(x, table):
    out = jnp.int32(table[-1])
    for i in range(len(table) - 1):
        out = jnp.where(x == i, jnp.int32(table[i]), out)
    return out


def _slow_sc(frames, idx):
    B, T, C, H, W = frames.shape
    S = _SLOW_FRAMES
    n_rows = B * S
    nw = _NUM_CORES * _NUM_SUBCORES
    rows_per_w = -(-n_rows // nw)
    HB = 128
    n_h = H // HB
    chunks = [(c, h * HB) for c in range(C) for h in range(n_h)]

    mesh = plsc.VectorSubcoreMesh(core_axis_name="c", subcore_axis_name="s")

    @functools.partial(
        pl.kernel,
        mesh=mesh,
        out_type=jax.ShapeDtypeStruct((B, S, C, H, W), jnp.float32),
        scratch_types=[
            pltpu.VMEM((2, HB, W), jnp.float32),
            pltpu.SemaphoreType.DMA,
            pltpu.SemaphoreType.DMA,
        ],
    )
    def gather_k(in_hbm, out_hbm, buf, rsem, wsem):
        wid = lax.axis_index("s") * _NUM_CORES + lax.axis_index("c")
        reads, writes = [], []
        for r in range(rows_per_w):
            i = wid * rows_per_w + r
            b = i // S
            j = i - b * S
            t = _select_const(j, idx)
            for (c, h0) in chunks:
                k = len(reads)
                src = in_hbm.at[b, t, c, pl.ds(h0, HB)]
                dst = out_hbm.at[b, j, c, pl.ds(h0, HB)]
                reads.append(pltpu.make_async_copy(src, buf.at[k % 2], rsem))
                writes.append(pltpu.make_async_copy(buf.at[k % 2], dst, wsem))
        n = len(reads)
        reads[0].start()
        for k in range(n):
            reads[k].wait()
            writes[k].start()
            if k + 1 < n:
                if k >= 1:
                    writes[k - 1].wait()
                reads[k + 1].start()
        writes[n - 1].wait()

    return gather_k(frames)


def _fast_tc(frames):
    B, T, C, H, W = frames.shape
    Tb = 16 if T % 16 == 0 else 1

    def body(in_ref, out_ref):
        out_ref[...] = in_ref[...]

    return pl.pallas_call(
        body,
        grid=(B, T // Tb),
        in_specs=[pl.BlockSpec((1, Tb, C, H, W), lambda b, t: (b, t, 0, 0, 0))],
        out_specs=pl.BlockSpec((1, Tb, C, H, W), lambda b, t: (b, t, 0, 0, 0)),
        out_shape=jax.ShapeDtypeStruct((B, T, C, H, W), frames.dtype),
    )(frames)


def kernel(frames):
    T = frames.shape[1]
    idx = [int(v) for v in np.linspace(0, T - 1, _SLOW_FRAMES).astype(np.int32)]
    slow = _slow_sc(frames, idx)
    fast = _fast_tc(frames)
    return (slow, fast)


# speedup vs baseline: 1.2362x; 1.2362x over previous
import jax
import jax.numpy as jnp
import numpy as np
from jax.experimental import pallas as pl

_SLOW_FRAMES = 8


def _pick_tb(T, S, idx):
    for tb in (16, 8, 4, 2, 1):
        if T % tb or S % (T // tb):
            continue
        per = S // (T // tb)
        counts = [sum(1 for v in idx if blk * tb <= v < (blk + 1) * tb)
                  for blk in range(T // tb)]
        if all(c == per for c in counts):
            return tb, per
    return 1, 1


def _select_const(x, table):
    out = jnp.int32(table[-1])
    for i in range(len(table) - 1):
        out = jnp.where(x == i, jnp.int32(table[i]), out)
    return out


def kernel(frames):
    B, T, C, H, W = frames.shape
    S = _SLOW_FRAMES
    idx = [int(v) for v in np.linspace(0, T - 1, S).astype(np.int32)]
    Tb, per = _pick_tb(T, S, idx)
    nblk = T // Tb
    offs = [[idx[tb * per + k] - tb * Tb for tb in range(nblk)]
            for k in range(per)]

    def body(in_ref, slow_ref, fast_ref):
        fast_ref[...] = in_ref[...]
        tb = pl.program_id(1)
        for k in range(per):
            off = _select_const(tb, offs[k])
            slow_ref[0, k] = in_ref[0, off]

    slow, fast = pl.pallas_call(
        body,
        grid=(B, nblk),
        in_specs=[
            pl.BlockSpec((1, Tb, C, H, W), lambda b, t: (b, t, 0, 0, 0)),
        ],
        out_specs=[
            pl.BlockSpec((1, per, C, H, W), lambda b, t: (b, t, 0, 0, 0)),
            pl.BlockSpec((1, Tb, C, H, W), lambda b, t: (b, t, 0, 0, 0)),
        ],
        out_shape=[
            jax.ShapeDtypeStruct((B, S, C, H, W), frames.dtype),
            jax.ShapeDtypeStruct((B, T, C, H, W), frames.dtype),
        ],
    )(frames)
    return (slow, fast)


# speedup vs baseline: 1.2370x; 1.0006x over previous
import jax
import jax.numpy as jnp
import numpy as np
from jax.experimental import pallas as pl

_SLOW_FRAMES = 8


def _pick_tb(T, S, idx):
    for tb in (16, 8, 4, 2, 1):
        if T % tb or S % (T // tb):
            continue
        per = S // (T // tb)
        counts = [sum(1 for v in idx if blk * tb <= v < (blk + 1) * tb)
                  for blk in range(T // tb)]
        if all(c == per for c in counts):
            return tb, per
    return None, None


def _select_const(x, table):
    out = jnp.int32(table[-1])
    for i in range(len(table) - 1):
        out = jnp.where(x == i, jnp.int32(table[i]), out)
    return out


def _kernel_revisit(frames, idx):
    B, T, C, H, W = frames.shape
    S = _SLOW_FRAMES

    def body(in_ref, slow_ref, fast_ref):
        data = in_ref[...]
        fast_ref[...] = data
        t = pl.program_id(1)
        sel = jnp.bool_(False)
        for c in idx:
            sel = sel | (t == c)

        @pl.when(sel)
        def _():
            slow_ref[...] = data

    def slow_map(b, t):
        j = jnp.int32(-1)
        for c in idx:
            j = j + (t >= c).astype(jnp.int32)
        return (b, j, 0, 0, 0)

    return pl.pallas_call(
        body,
        grid=(B, T),
        in_specs=[
            pl.BlockSpec((1, 1, C, H, W), lambda b, t: (b, t, 0, 0, 0)),
        ],
        out_specs=[
            pl.BlockSpec((1, 1, C, H, W), slow_map),
            pl.BlockSpec((1, 1, C, H, W), lambda b, t: (b, t, 0, 0, 0)),
        ],
        out_shape=[
            jax.ShapeDtypeStruct((B, S, C, H, W), frames.dtype),
            jax.ShapeDtypeStruct((B, T, C, H, W), frames.dtype),
        ],
    )(frames)


def kernel(frames):
    B, T, C, H, W = frames.shape
    S = _SLOW_FRAMES
    idx = [int(v) for v in np.linspace(0, T - 1, S).astype(np.int32)]
    Tb, per = _pick_tb(T, S, idx)
    if Tb is None:
        slow, fast = _kernel_revisit(frames, idx)
        return (slow, fast)
    nblk = T // Tb
    offs = [[idx[tb * per + k] - tb * Tb for tb in range(nblk)]
            for k in range(per)]

    def body(in_ref, slow_ref, fast_ref):
        fast_ref[...] = in_ref[...]
        tb = pl.program_id(1)
        for k in range(per):
            off = _select_const(tb, offs[k])
            slow_ref[0, k] = in_ref[0, off]

    slow, fast = pl.pallas_call(
        body,
        grid=(B, nblk),
        in_specs=[
            pl.BlockSpec((1, Tb, C, H, W), lambda b, t: (b, t, 0, 0, 0)),
        ],
        out_specs=[
            pl.BlockSpec((1, per, C, H, W), lambda b, t: (b, t, 0, 0, 0)),
            pl.BlockSpec((1, Tb, C, H, W), lambda b, t: (b, t, 0, 0, 0)),
        ],
        out_shape=[
            jax.ShapeDtypeStruct((B, S, C, H, W), frames.dtype),
            jax.ShapeDtypeStruct((B, T, C, H, W), frames.dtype),
        ],
    )(frames)
    return (slow, fast)
